# Initial kernel scaffold; baseline (speedup 1.0000x reference)
import jax
import jax.numpy as jnp
from jax.experimental import pallas as pl


def kernel(x, edge_index, W1, b1, W2, b2, W3, b3, W4, b4):
    raise NotImplementedError("write your pallas kernel here")


# speedup vs baseline: 6.1671x; 6.1671x over previous
import functools

import jax
import jax.numpy as jnp
from jax import lax
from jax.experimental import pallas as pl
from jax.experimental.pallas import tpu as pltpu
from jax.experimental.pallas import tpu_sc as plsc

_NC = 2
_NS = 16
_NW = _NC * _NS
_C = 128
_W = 128


def _zero_acc_slice(zbuf, acc, zr, base, rpt):
  @pl.loop(0, zr)
  def _zero_rows(i):
    for l in range(_W // 16):
      zbuf[i, pl.ds(l * 16, 16)] = jnp.zeros((16,), jnp.float32)
  off = 0
  while off < rpt:
    nrows = min(zr, rpt - off)
    pltpu.sync_copy(zbuf.at[pl.ds(0, nrows)], acc.at[pl.ds(base + off, nrows)])
    off += nrows


@functools.lru_cache(maxsize=None)
def _make_agg(n_acc: int, ch: int):
  assert n_acc % (8 * _NS) == 0
  rpt = n_acc // _NS
  zr = min(rpt, 32)

  def body(table, srcr, dstr, out, idx_s, idx_d, rows0, rows1, zbuf, acc,
           sem0, sem1):
    c = lax.axis_index("c")
    s = lax.axis_index("s")
    wid = c * _NS + s

    pltpu.sync_copy(srcr.at[pl.ds(wid * ch, ch)], idx_s)
    pltpu.sync_copy(dstr.at[pl.ds(wid * ch, ch)], idx_d)
    _zero_acc_slice(zbuf, acc, zr, s * rpt, rpt)
    plsc.subcore_barrier()

    pltpu.async_copy(table.at[idx_s.at[0]], rows0, sem0)

    @pl.loop(0, ch // 2)
    def _pipe(k):
      j0 = k * 2
      pltpu.async_copy(table.at[idx_s.at[j0 + 1]], rows1, sem1)
      pltpu.make_async_copy(table.at[idx_s.at[0]], rows0, sem0).wait()
      pltpu.sync_copy(rows0, acc.at[idx_d.at[j0]], add=True)

      @pl.when(k < ch // 2 - 1)
      def _():
        pltpu.async_copy(table.at[idx_s.at[j0 + 2]], rows0, sem0)

      pltpu.make_async_copy(table.at[idx_s.at[0]], rows1, sem1).wait()
      pltpu.sync_copy(rows1, acc.at[idx_d.at[j0 + 1]], add=True)

    plsc.subcore_barrier()
    pltpu.sync_copy(acc.at[pl.ds(s * rpt, rpt)],
                    out.at[c, pl.ds(s * rpt, rpt)])

  return pl.kernel(
      body,
      out_type=jax.ShapeDtypeStruct((_NC, n_acc, _W), jnp.float32),
      mesh=plsc.VectorSubcoreMesh(core_axis_name="c", subcore_axis_name="s",
                                  num_cores=_NC, num_subcores=_NS),
      scratch_types=[
          pltpu.VMEM((ch, _C), jnp.int32),
          pltpu.VMEM((ch, _C), jnp.int32),
          pltpu.VMEM((_C, _W), jnp.float32),
          pltpu.VMEM((_C, _W), jnp.float32),
          pltpu.VMEM((zr, _W), jnp.float32),
          pltpu.VMEM_SHARED((n_acc, _W), jnp.float32),
          pltpu.SemaphoreType.DMA,
          pltpu.SemaphoreType.DMA,
      ],
  )


@functools.lru_cache(maxsize=None)
def _make_deg(n_acc: int, ch: int):
  assert n_acc % (8 * _NS) == 0
  rpt = n_acc // _NS
  zr = min(rpt, 16384 // _W)

  def body(dstr, out, idx_d, ones, zbuf, acc):
    c = lax.axis_index("c")
    s = lax.axis_index("s")
    wid = c * _NS + s

    pltpu.sync_copy(dstr.at[pl.ds(wid * ch, ch)], idx_d)

    @pl.loop(0, _C)
    def _fill_ones(i):
      for l in range(_W // 16):
        ones[i, pl.ds(l * 16, 16)] = jnp.ones((16,), jnp.float32)

    _zero_acc_slice(zbuf, acc, zr, s * rpt, rpt)
    plsc.subcore_barrier()

    @pl.loop(0, ch)
    def _scat(j):
      pltpu.sync_copy(ones, acc.at[idx_d.at[j]], add=True)

    plsc.subcore_barrier()
    pltpu.sync_copy(acc.at[pl.ds(s * rpt, rpt)],
                    out.at[c, pl.ds(s * rpt, rpt)])

  return pl.kernel(
      body,
      out_type=jax.ShapeDtypeStruct((_NC, n_acc, _W), jnp.float32),
      mesh=plsc.VectorSubcoreMesh(core_axis_name="c", subcore_axis_name="s",
                                  num_cores=_NC, num_subcores=_NS),
      scratch_types=[
          pltpu.VMEM((ch, _C), jnp.int32),
          pltpu.VMEM((_C, _W), jnp.float32),
          pltpu.VMEM((zr, _W), jnp.float32),
          pltpu.VMEM_SHARED((n_acc, _W), jnp.float32),
      ],
  )


def _prep_body(d_ref, x_ref, w_ref, dis_ref, p_ref):
  deg = d_ref[0, :, 0:1] + d_ref[1, :, 0:1] + 1.0
  dis = lax.rsqrt(deg)
  dis_ref[...] = dis
  g = jnp.dot(x_ref[...], w_ref[...], preferred_element_type=jnp.float32)
  p_ref[...] = g * dis


def _tc_prep(deg2, x, w1, br=1000):
  n, d = x.shape
  h = w1.shape[1]
  return pl.pallas_call(
      _prep_body,
      grid=(n // br,),
      in_specs=[
          pl.BlockSpec((_NC, br, _W), lambda i: (0, i, 0)),
          pl.BlockSpec((br, d), lambda i: (i, 0)),
          pl.BlockSpec((d, h), lambda i: (0, 0)),
      ],
      out_specs=[
          pl.BlockSpec((br, 1), lambda i: (i, 0)),
          pl.BlockSpec((br, h), lambda i: (i, 0)),
      ],
      out_shape=[
          jax.ShapeDtypeStruct((n, 1), jnp.float32),
          jax.ShapeDtypeStruct((n, h), jnp.float32),
      ],
  )(deg2, x, w1)


def _mid_body(s_ref, p_ref, dis_ref, b_ref, w_ref, pn_ref):
  dis = dis_ref[...]
  hid = dis * (s_ref[0] + s_ref[1] + p_ref[...]) + b_ref[...]
  hid = jnp.maximum(hid, 0.0)
  pn_ref[...] = jnp.dot(hid, w_ref[...],
                        preferred_element_type=jnp.float32) * dis


def _tc_mid(s2, p, dis, b, w, br=1000):
  n, h = p.shape
  ho = w.shape[1]
  return pl.pallas_call(
      _mid_body,
      grid=(n // br,),
      in_specs=[
          pl.BlockSpec((_NC, br, h), lambda i: (0, i, 0)),
          pl.BlockSpec((br, h), lambda i: (i, 0)),
          pl.BlockSpec((br, 1), lambda i: (i, 0)),
          pl.BlockSpec((1, h), lambda i: (0, 0)),
          pl.BlockSpec((h, ho), lambda i: (0, 0)),
      ],
      out_specs=pl.BlockSpec((br, ho), lambda i: (i, 0)),
      out_shape=jax.ShapeDtypeStruct((n, ho), jnp.float32),
  )(s2, p, dis, b, w)


def _q_body(s_ref, p_ref, dis_ref, b_ref, q_ref):
  dis = dis_ref[...]
  hid = dis * (s_ref[0] + s_ref[1] + p_ref[...]) + b_ref[...]
  q_ref[...] = jnp.maximum(hid, 0.0) * dis


def _tc_q(s3, p, dis, b, br=1000):
  n, h = p.shape
  return pl.pallas_call(
      _q_body,
      grid=(n // br,),
      in_specs=[
          pl.BlockSpec((_NC, br, h), lambda i: (0, i, 0)),
          pl.BlockSpec((br, h), lambda i: (i, 0)),
          pl.BlockSpec((br, 1), lambda i: (i, 0)),
          pl.BlockSpec((1, h), lambda i: (0, 0)),
      ],
      out_specs=pl.BlockSpec((br, h), lambda i: (i, 0)),
      out_shape=jax.ShapeDtypeStruct((n, h), jnp.float32),
  )(s3, p, dis, b)


def _out_body(t_ref, q_ref, dis_ref, w_ref, b_ref, o_ref):
  tot = t_ref[0] + t_ref[1] + q_ref[...]
  y = jnp.dot(tot, w_ref[...], preferred_element_type=jnp.float32)
  o_ref[...] = dis_ref[...] * y + b_ref[0, 0]


def _tc_out(t4, q, dis, w4, b4, br=1000):
  n, h = q.shape
  return pl.pallas_call(
      _out_body,
      grid=(n // br,),
      in_specs=[
          pl.BlockSpec((_NC, br, h), lambda i: (0, i, 0)),
          pl.BlockSpec((br, h), lambda i: (i, 0)),
          pl.BlockSpec((br, 1), lambda i: (i, 0)),
          pl.BlockSpec((h, 1), lambda i: (0, 0)),
          pl.BlockSpec((1, 1), lambda i: (0, 0)),
      ],
      out_specs=pl.BlockSpec((br, 1), lambda i: (i, 0)),
      out_shape=jax.ShapeDtypeStruct((n, 1), jnp.float32),
  )(t4, q, dis, w4, b4)


def kernel(x, edge_index, W1, b1, W2, b2, W3, b3, W4, b4):
  n, _ = x.shape
  h = W1.shape[1]
  e = edge_index.shape[1]
  ch = -(-e // (_NW * _C))
  e_pad = _NW * ch * _C
  n_acc = -(-n // (8 * _NS)) * (8 * _NS)
  if n_acc == n:
    n_acc += 8 * _NS

  src = edge_index[0]
  dst = edge_index[1]
  pad = e_pad - e
  srcr = jnp.concatenate(
      [src, jnp.zeros((pad,), src.dtype)]).reshape(_NW * ch, _C)
  dstr = jnp.concatenate(
      [dst, jnp.full((pad,), n, dst.dtype)]).reshape(_NW * ch, _C)

  agg = _make_agg(n_acc, ch)
  deg2 = _make_deg(n_acc, ch)(dstr)

  dis, p1 = _tc_prep(deg2, x, W1)
  s1 = agg(p1, srcr, dstr)
  p2 = _tc_mid(s1, p1, dis, b1.reshape(1, h), W2)
  s2 = agg(p2, srcr, dstr)
  p3 = _tc_mid(s2, p2, dis, b2.reshape(1, h), W3)
  s3 = agg(p3, srcr, dstr)
  q = _tc_q(s3, p3, dis, b3.reshape(1, h))
  t4 = agg(q, srcr, dstr)
  return _tc_out(t4, q, dis, W4, b4.reshape(1, 1))


# speedup vs baseline: 18.6507x; 3.0242x over previous
import functools

import jax
import jax.numpy as jnp
from jax import lax
from jax.experimental import pallas as pl
from jax.experimental.pallas import tpu as pltpu
from jax.experimental.pallas import tpu_sc as plsc

_NC = 2
_NS = 16
_NW = _NC * _NS
_C = 128
_W = 128


def _zero_acc_slice(zbuf, acc, zr, base, rpt):
  @pl.loop(0, zr)
  def _zero_rows(i):
    for l in range(_W // 16):
      zbuf[i, pl.ds(l * 16, 16)] = jnp.zeros((16,), jnp.float32)
  off = 0
  while off < rpt:
    nrows = min(zr, rpt - off)
    pltpu.sync_copy(zbuf.at[pl.ds(0, nrows)], acc.at[pl.ds(base + off, nrows)])
    off += nrows


@functools.lru_cache(maxsize=None)
def _make_agg(n_acc: int, ch: int):
  assert n_acc % (8 * _NS) == 0
  rpt = n_acc // _NS
  zr = min(rpt, 32)

  def body(table, srcr, dstr, out, idx_s, idx_d, rows0, rows1, zbuf, acc,
           sem0, sem1):
    c = lax.axis_index("c")
    s = lax.axis_index("s")
    wid = c * _NS + s

    pltpu.sync_copy(srcr.at[pl.ds(wid * ch, ch)], idx_s)
    pltpu.sync_copy(dstr.at[pl.ds(wid * ch, ch)], idx_d)
    _zero_acc_slice(zbuf, acc, zr, s * rpt, rpt)
    plsc.subcore_barrier()

    pltpu.async_copy(table.at[idx_s.at[0]], rows0, sem0)

    @pl.loop(0, ch // 2)
    def _pipe(k):
      j0 = k * 2
      pltpu.async_copy(table.at[idx_s.at[j0 + 1]], rows1, sem1)
      pltpu.make_async_copy(table.at[idx_s.at[0]], rows0, sem0).wait()
      pltpu.sync_copy(rows0, acc.at[idx_d.at[j0]], add=True)

      @pl.when(k < ch // 2 - 1)
      def _():
        pltpu.async_copy(table.at[idx_s.at[j0 + 2]], rows0, sem0)

      pltpu.make_async_copy(table.at[idx_s.at[0]], rows1, sem1).wait()
      pltpu.sync_copy(rows1, acc.at[idx_d.at[j0 + 1]], add=True)

    plsc.subcore_barrier()
    pltpu.sync_copy(acc.at[pl.ds(s * rpt, rpt)],
                    out.at[c, pl.ds(s * rpt, rpt)])

  return pl.kernel(
      body,
      out_type=jax.ShapeDtypeStruct((_NC, n_acc, _W), jnp.float32),
      mesh=plsc.VectorSubcoreMesh(core_axis_name="c", subcore_axis_name="s",
                                  num_cores=_NC, num_subcores=_NS),
      scratch_types=[
          pltpu.VMEM((ch, _C), jnp.int32),
          pltpu.VMEM((ch, _C), jnp.int32),
          pltpu.VMEM((_C, _W), jnp.float32),
          pltpu.VMEM((_C, _W), jnp.float32),
          pltpu.VMEM((zr, _W), jnp.float32),
          pltpu.VMEM_SHARED((n_acc, _W), jnp.float32),
          pltpu.SemaphoreType.DMA,
          pltpu.SemaphoreType.DMA,
      ],
  )


@functools.lru_cache(maxsize=None)
def _make_deg(n_acc: int, ch: int):
  assert n_acc % (8 * _NS) == 0
  rpt = n_acc // _NS
  zr = min(rpt, 16384 // _W)

  def body(dstr, out, idx_d, ones, zbuf, acc):
    c = lax.axis_index("c")
    s = lax.axis_index("s")
    wid = c * _NS + s

    pltpu.sync_copy(dstr.at[pl.ds(wid * ch, ch)], idx_d)

    @pl.loop(0, _C)
    def _fill_ones(i):
      for l in range(_W // 16):
        ones[i, pl.ds(l * 16, 16)] = jnp.ones((16,), jnp.float32)

    _zero_acc_slice(zbuf, acc, zr, s * rpt, rpt)
    plsc.subcore_barrier()

    @pl.loop(0, ch)
    def _scat(j):
      pltpu.sync_copy(ones, acc.at[idx_d.at[j]], add=True)

    plsc.subcore_barrier()
    pltpu.sync_copy(acc.at[pl.ds(s * rpt, rpt)],
                    out.at[c, pl.ds(s * rpt, rpt)])

  return pl.kernel(
      body,
      out_type=jax.ShapeDtypeStruct((_NC, n_acc, _W), jnp.float32),
      mesh=plsc.VectorSubcoreMesh(core_axis_name="c", subcore_axis_name="s",
                                  num_cores=_NC, num_subcores=_NS),
      scratch_types=[
          pltpu.VMEM((ch, _C), jnp.int32),
          pltpu.VMEM((_C, _W), jnp.float32),
          pltpu.VMEM((zr, _W), jnp.float32),
          pltpu.VMEM_SHARED((n_acc, _W), jnp.float32),
      ],
  )


def _prep_body(d_ref, x_ref, w_ref, dis_ref, p_ref):
  deg = d_ref[0, :, 0:1] + d_ref[1, :, 0:1] + 1.0
  dis = lax.rsqrt(deg)
  dis_ref[...] = dis
  g = jnp.dot(x_ref[...], w_ref[...], preferred_element_type=jnp.float32)
  p_ref[...] = g * dis


def _tc_prep(deg2, x, w1, br=1000):
  n, d = x.shape
  h = w1.shape[1]
  return pl.pallas_call(
      _prep_body,
      grid=(n // br,),
      in_specs=[
          pl.BlockSpec((_NC, br, _W), lambda i: (0, i, 0)),
          pl.BlockSpec((br, d), lambda i: (i, 0)),
          pl.BlockSpec((d, h), lambda i: (0, 0)),
      ],
      out_specs=[
          pl.BlockSpec((br, 1), lambda i: (i, 0)),
          pl.BlockSpec((br, h), lambda i: (i, 0)),
      ],
      out_shape=[
          jax.ShapeDtypeStruct((n, 1), jnp.float32),
          jax.ShapeDtypeStruct((n, h), jnp.float32),
      ],
  )(deg2, x, w1)


def _mid_body(s_ref, p_ref, dis_ref, b_ref, w_ref, pn_ref):
  dis = dis_ref[...]
  hid = dis * (s_ref[0] + s_ref[1] + p_ref[...]) + b_ref[...]
  hid = jnp.maximum(hid, 0.0)
  pn_ref[...] = jnp.dot(hid, w_ref[...],
                        preferred_element_type=jnp.float32) * dis


def _tc_mid(s2, p, dis, b, w, br=1000):
  n, h = p.shape
  ho = w.shape[1]
  return pl.pallas_call(
      _mid_body,
      grid=(n // br,),
      in_specs=[
          pl.BlockSpec((_NC, br, h), lambda i: (0, i, 0)),
          pl.BlockSpec((br, h), lambda i: (i, 0)),
          pl.BlockSpec((br, 1), lambda i: (i, 0)),
          pl.BlockSpec((1, h), lambda i: (0, 0)),
          pl.BlockSpec((h, ho), lambda i: (0, 0)),
      ],
      out_specs=pl.BlockSpec((br, ho), lambda i: (i, 0)),
      out_shape=jax.ShapeDtypeStruct((n, ho), jnp.float32),
  )(s2, p, dis, b, w)


def _q_body(s_ref, p_ref, dis_ref, b_ref, q_ref):
  dis = dis_ref[...]
  hid = dis * (s_ref[0] + s_ref[1] + p_ref[...]) + b_ref[...]
  q_ref[...] = jnp.maximum(hid, 0.0) * dis


def _tc_q(s3, p, dis, b, br=1000):
  n, h = p.shape
  return pl.pallas_call(
      _q_body,
      grid=(n // br,),
      in_specs=[
          pl.BlockSpec((_NC, br, h), lambda i: (0, i, 0)),
          pl.BlockSpec((br, h), lambda i: (i, 0)),
          pl.BlockSpec((br, 1), lambda i: (i, 0)),
          pl.BlockSpec((1, h), lambda i: (0, 0)),
      ],
      out_specs=pl.BlockSpec((br, h), lambda i: (i, 0)),
      out_shape=jax.ShapeDtypeStruct((n, h), jnp.float32),
  )(s3, p, dis, b)


def _out_body(t_ref, q_ref, dis_ref, w_ref, b_ref, o_ref):
  tot = t_ref[0] + t_ref[1] + q_ref[...]
  y = jnp.dot(tot, w_ref[...], preferred_element_type=jnp.float32)
  o_ref[...] = dis_ref[...] * y + b_ref[0, 0]


def _tc_out(t4, q, dis, w4, b4, br=1000):
  n, h = q.shape
  return pl.pallas_call(
      _out_body,
      grid=(n // br,),
      in_specs=[
          pl.BlockSpec((_NC, br, h), lambda i: (0, i, 0)),
          pl.BlockSpec((br, h), lambda i: (i, 0)),
          pl.BlockSpec((br, 1), lambda i: (i, 0)),
          pl.BlockSpec((h, 1), lambda i: (0, 0)),
          pl.BlockSpec((1, 1), lambda i: (0, 0)),
      ],
      out_specs=pl.BlockSpec((br, 1), lambda i: (i, 0)),
      out_shape=jax.ShapeDtypeStruct((n, 1), jnp.float32),
  )(t4, q, dis, w4, b4)


def kernel(x, edge_index, W1, b1, W2, b2, W3, b3, W4, b4):
  n, _ = x.shape
  h = W1.shape[1]
  e = edge_index.shape[1]
  ch = -(-e // (_NW * _C))
  e_pad = _NW * ch * _C
  n_acc = -(-n // (8 * _NS)) * (8 * _NS)
  if n_acc == n:
    n_acc += 8 * _NS

  src = edge_index[0]
  dst = edge_index[1]
  pad = e_pad - e
  pad_src = jnp.arange(pad, dtype=src.dtype) % n
  pad_dst = n + jnp.arange(pad, dtype=dst.dtype) % (n_acc - n)
  srcr = jnp.concatenate([src, pad_src]).reshape(_NW * ch, _C)
  dstr = jnp.concatenate([dst, pad_dst]).reshape(_NW * ch, _C)

  agg = _make_agg(n_acc, ch)
  deg2 = _make_deg(n_acc, ch)(dstr)

  dis, p1 = _tc_prep(deg2, x, W1)
  s1 = agg(p1, srcr, dstr)
  p2 = _tc_mid(s1, p1, dis, b1.reshape(1, h), W2)
  s2 = agg(p2, srcr, dstr)
  p3 = _tc_mid(s2, p2, dis, b2.reshape(1, h), W3)
  s3 = agg(p3, srcr, dstr)
  q = _tc_q(s3, p3, dis, b3.reshape(1, h))
  t4 = agg(q, srcr, dstr)
  return _tc_out(t4, q, dis, W4, b4.reshape(1, 1))
